# Initial kernel scaffold; baseline (speedup 1.0000x reference)
import jax
import jax.numpy as jnp
from jax.experimental import pallas as pl


def kernel(bbox_pred, conf_pred, anchors, target_boxes, target_labels, conf_weight, bbox_weight):
    raise NotImplementedError("write your pallas kernel here")


# speedup vs baseline: 32.8015x; 32.8015x over previous
import jax
import jax.numpy as jnp
from jax import lax
from jax.experimental import pallas as pl
from jax.experimental.pallas import tpu as pltpu

POS_T = 0.5
NEG_T = 0.4
ALPHA = 0.25
GAMMA = 2.0
N_BISECT = 30


def _loss_body(conf_ref, bboxp_ref, anch_ref, tb_ref, tl_ref, out_ref, nf_ref):
    C, A = conf_ref.shape
    T = tb_ref.shape[0]
    f32 = jnp.float32
    i32 = jnp.int32

    ax1 = anch_ref[0:1, :]
    ay1 = anch_ref[1:2, :]
    ax2 = anch_ref[2:3, :]
    ay2 = anch_ref[3:4, :]
    tx1 = tb_ref[:, 0:1]
    ty1 = tb_ref[:, 1:2]
    tx2 = tb_ref[:, 2:3]
    ty2 = tb_ref[:, 3:4]

    ix1 = jnp.maximum(ax1, tx1)
    iy1 = jnp.maximum(ay1, ty1)
    ix2 = jnp.minimum(ax2, tx2)
    iy2 = jnp.minimum(ay2, ty2)
    inter = jnp.maximum(ix2 - ix1, 0.0) * jnp.maximum(iy2 - iy1, 0.0)
    area_a = (ax2 - ax1) * (ay2 - ay1)
    area_t = (tx2 - tx1) * (ty2 - ty1)
    union = area_a + area_t - inter
    iou = inter / (union + 1e-6)

    t_iota = lax.broadcasted_iota(i32, (T, A), 0)
    a_iota = lax.broadcasted_iota(i32, (T, A), 1)

    max_iou = jnp.max(iou, axis=0, keepdims=True)
    best_t = jnp.min(jnp.where(iou == max_iou, t_iota, T), axis=0, keepdims=True)

    col_max = jnp.max(iou, axis=1, keepdims=True)
    ba = jnp.min(jnp.where(iou == col_max, a_iota, A), axis=1, keepdims=True)
    forced = jnp.max(jnp.where(a_iota == ba, 1, 0), axis=0, keepdims=True) > 0

    pos = (max_iou >= POS_T) | forced
    neg = (max_iou < NEG_T) & jnp.logical_not(forced)
    num_pos = jnp.sum(pos.astype(i32))
    n_neg = jnp.sum(neg.astype(i32))

    eq_t = best_t == t_iota
    tlv = tl_ref[:, 0:1]
    ml = jnp.sum(jnp.where(eq_t, tlv, 0), axis=0, keepdims=True)

    x = conf_ref[...]
    m = jnp.max(x, axis=0, keepdims=True)
    e = jnp.exp(x - m)
    s = jnp.sum(e, axis=0, keepdims=True)
    probs = e / s
    c_iota = lax.broadcasted_iota(i32, (C, A), 0)
    is_tgt = pos & (c_iota == ml)
    pt = jnp.where(is_tgt, probs, 1.0 - probs)
    alpha_f = jnp.where(is_tgt, ALPHA, 1.0 - ALPHA)
    omp = 1.0 - pt
    focal = -alpha_f * omp * omp * jnp.log(jnp.maximum(pt, 1e-6))

    pos_loss = jnp.sum(jnp.where(pos, focal, 0.0))
    focal_mean = jnp.sum(focal) / f32(A * C)

    nf_ref[...] = jnp.where(neg, focal, -1.0)

    k1 = jnp.minimum(num_pos * 3, n_neg)
    k2 = jnp.minimum(jnp.int32(100), n_neg * C)
    k1f = k1.astype(f32)
    k2f = k2.astype(f32)

    def bisect_step(_, carry):
        lo1, hi1, lo2, hi2 = carry
        mid1 = 0.5 * (lo1 + hi1)
        mid2 = 0.5 * (lo2 + hi2)
        nf = nf_ref[...]
        c1 = jnp.sum((nf > mid1).astype(f32))
        c2 = jnp.sum((nf > mid2).astype(f32))
        lo1, hi1 = jnp.where(c1 > k1f, mid1, lo1), jnp.where(c1 > k1f, hi1, mid1)
        lo2, hi2 = jnp.where(c2 > k2f, mid2, lo2), jnp.where(c2 > k2f, hi2, mid2)
        return lo1, hi1, lo2, hi2

    lo0 = f32(-0.5)
    hi0 = f32(11.0)
    _, tau1, _, tau2 = lax.fori_loop(0, N_BISECT, bisect_step,
                                     (lo0, hi0, lo0, hi0))

    nf = nf_ref[...]
    gt1 = nf > tau1
    s1 = jnp.sum(jnp.where(gt1, nf, 0.0))
    c1 = jnp.sum(gt1.astype(f32))
    neg_sum = s1 + tau1 * (k1f - c1)
    gt2 = nf > tau2
    s2 = jnp.sum(jnp.where(gt2, nf, 0.0))
    c2 = jnp.sum(gt2.astype(f32))
    top_sum = s2 + tau2 * (k2f - c2)

    px1 = bboxp_ref[0:1, :]
    py1 = bboxp_ref[1:2, :]
    px2 = bboxp_ref[2:3, :]
    py2 = bboxp_ref[3:4, :]
    mx1 = jnp.sum(jnp.where(eq_t, tx1, 0.0), axis=0, keepdims=True)
    my1 = jnp.sum(jnp.where(eq_t, ty1, 0.0), axis=0, keepdims=True)
    mx2 = jnp.sum(jnp.where(eq_t, tx2, 0.0), axis=0, keepdims=True)
    my2 = jnp.sum(jnp.where(eq_t, ty2, 0.0), axis=0, keepdims=True)

    gx1 = jnp.maximum(px1, mx1)
    gy1 = jnp.maximum(py1, my1)
    gx2 = jnp.minimum(px2, mx2)
    gy2 = jnp.minimum(py2, my2)
    ginter = jnp.maximum(gx2 - gx1, 0.0) * jnp.maximum(gy2 - gy1, 0.0)
    parea = (px2 - px1) * (py2 - py1)
    marea = (mx2 - mx1) * (my2 - my1)
    gunion = parea + marea - ginter
    giou_iou = ginter / (gunion + 1e-6)
    ex1 = jnp.minimum(px1, mx1)
    ey1 = jnp.minimum(py1, my1)
    ex2 = jnp.maximum(px2, mx2)
    ey2 = jnp.maximum(py2, my2)
    enc = (ex2 - ex1) * (ey2 - ey1)
    giou = giou_iou - (enc - gunion) / (enc + 1e-6)
    iou_l = 1.0 - giou

    def sl1(d):
        ad = jnp.abs(d)
        return jnp.where(ad < 1.0, 0.5 * ad * ad, ad - 0.5)

    l1 = (sl1(px1 - mx1) + sl1(py1 - my1) + sl1(px2 - mx2) + sl1(py2 - my2)) * 0.25
    per_anchor = iou_l + 0.5 * l1
    bbox_sum = jnp.sum(jnp.where(pos, per_anchor, 0.0))

    npf = jnp.maximum(num_pos, 1).astype(f32)
    denom1 = jnp.maximum(num_pos + k1, 1).astype(f32)
    conf_loss_pos = jnp.where(n_neg > 0, (pos_loss + neg_sum) / denom1,
                              pos_loss / npf)
    conf_loss_neg = jnp.where(n_neg * C > 0,
                              top_sum / jnp.maximum(k2, 1).astype(f32),
                              focal_mean)
    has_pos = num_pos > 0
    conf_i = jnp.where(has_pos, conf_loss_pos, conf_loss_neg)
    bbox_i = jnp.where(has_pos, bbox_sum / npf, 0.0)

    row = lax.broadcasted_iota(i32, (8, 128), 0)
    out_ref[...] = jnp.where(row == 0, conf_i, jnp.where(row == 1, bbox_i, 0.0))


def kernel(bbox_pred, conf_pred, anchors, target_boxes, target_labels,
           conf_weight=1.0, bbox_weight=1.0):
    B, A, C = conf_pred.shape
    T = target_boxes.shape[1]

    conf_t = jnp.transpose(conf_pred, (0, 2, 1))
    bbox_t = jnp.transpose(bbox_pred, (0, 2, 1))
    anch_t = jnp.transpose(anchors, (1, 0))
    tl3 = target_labels.astype(jnp.int32).reshape(B, T, 1)

    out = pl.pallas_call(
        _loss_body,
        grid=(B,),
        in_specs=[
            pl.BlockSpec((None, C, A), lambda i: (i, 0, 0)),
            pl.BlockSpec((None, 4, A), lambda i: (i, 0, 0)),
            pl.BlockSpec((4, A), lambda i: (0, 0)),
            pl.BlockSpec((None, T, 4), lambda i: (i, 0, 0)),
            pl.BlockSpec((None, T, 1), lambda i: (i, 0, 0)),
        ],
        out_specs=pl.BlockSpec((None, 8, 128), lambda i: (i, 0, 0)),
        out_shape=jax.ShapeDtypeStruct((B, 8, 128), jnp.float32),
        scratch_shapes=[pltpu.VMEM((C, A), jnp.float32)],
    )(conf_t, bbox_t, anch_t, target_boxes, tl3)

    conf_loss = jnp.mean(out[:, 0, 0])
    bbox_loss = jnp.mean(out[:, 1, 0])
    total_loss = conf_weight * conf_loss + bbox_weight * bbox_loss
    return total_loss, conf_loss, bbox_loss
